# baseline (speedup 1.0000x reference)
import functools
import math

import numpy as np

import jax
import jax.numpy as jnp
from jax import lax
from jax.experimental import pallas as pl
from jax.experimental.pallas import tpu as pltpu
from jax.experimental.pallas import tpu_sc as plsc

D_MODEL = 768
CONTEXT_LEN = 2048
LANES = 16

CHUNK = 16
NSET = 2
LOOK = 1


def _position_encoding(context_length, d_model):
    position = np.arange(0, context_length, dtype=np.float32)[:, None]
    div_term = np.exp(
        np.arange(0, d_model, 2).astype(np.float32) * (-math.log(10000.0) / d_model)
    )
    pe = np.zeros((context_length, d_model), dtype=np.float32)
    pe[:, 0::2] = np.sin(position * div_term)
    pe[:, 1::2] = np.cos(position * div_term)
    return jnp.asarray(pe)


def kernel(x, W):
    B, T = x.shape
    V, D = W.shape
    N = B * T
    pe = _position_encoding(CONTEXT_LEN, D_MODEL)[:T]

    NC, NS = 2, 16
    NW = NC * NS
    T_BLK = T // NW
    G = T_BLK // CHUNK

    x_flat = x.reshape(N).astype(jnp.int32)
    mesh = plsc.VectorSubcoreMesh(core_axis_name="c", subcore_axis_name="s")

    scratch = [pltpu.VMEM((B, T_BLK), jnp.int32)]
    scratch += [pltpu.VMEM((CHUNK, D), jnp.float32) for _ in range(NSET * (B + 1))]
    scratch += [pltpu.SemaphoreType.DMA for _ in range(3 * NSET)]

    @functools.partial(
        pl.kernel,
        out_type=jax.ShapeDtypeStruct((N, D), jnp.float32),
        mesh=mesh,
        scratch_types=scratch,
    )
    def emb(x_hbm, w_hbm, pe_hbm, out_hbm, idx_v, *rest):
        dbufs = [rest[s * B:(s + 1) * B] for s in range(NSET)]
        pbufs = rest[NSET * B:NSET * (B + 1)]
        base = NSET * (B + 1)
        gsems = rest[base:base + NSET]
        ssems = rest[base + NSET:base + 2 * NSET]
        fsems = rest[base + 2 * NSET:base + 3 * NSET]
        wid = lax.axis_index("s") * NC + lax.axis_index("c")
        t0 = wid * T_BLK

        idx_cps = [
            pltpu.async_copy(x_hbm.at[pl.ds(b * T + t0, T_BLK)],
                             idx_v.at[b], gsems[b % NSET])
            for b in range(B)
        ]
        for cp in idx_cps:
            cp.wait()

        def group_start(g):
            s = g % NSET
            cps = [pltpu.async_copy(
                pe_hbm.at[pl.ds(t0 + g * CHUNK, CHUNK)], pbufs[s], fsems[s])]
            cps += [pltpu.async_copy(
                w_hbm.at[idx_v.at[b, pl.ds(g * CHUNK, CHUNK)]],
                dbufs[s][b], gsems[s]) for b in range(B)]
            return cps

        gathers = [None] * NSET
        stores = [None] * NSET
        for g in range(min(LOOK + 1, G)):
            gathers[g % NSET] = group_start(g)

        for g in range(G):
            s = g % NSET
            for cp in gathers[s]:
                cp.wait()

            @pl.loop(0, CHUNK)
            def _(r):
                for j in range(D // LANES):
                    jslc = pl.ds(j * LANES, LANES)
                    pv = pbufs[s].at[pl.ds(r, 1), jslc][...]
                    for b in range(B):
                        plsc.addupdate(dbufs[s][b].at[pl.ds(r, 1), jslc], pv)

            stores[s] = [pltpu.async_copy(
                dbufs[s][b], out_hbm.at[pl.ds(b * T + t0 + g * CHUNK, CHUNK)],
                ssems[s]) for b in range(B)]
            ga = g + LOOK + 1
            if ga < G:
                sa = ga % NSET
                if ga >= NSET:
                    for cp in stores[sa]:
                        cp.wait()
                gathers[sa] = group_start(ga)
        for g in range(max(0, G - NSET), G):
            if stores[g % NSET] is not None:
                for cp in stores[g % NSET]:
                    cp.wait()

    out = emb(x_flat, W, pe)
    return out.reshape(B, T, D)
